# baseline (speedup 1.0000x reference)
import functools

import jax
import jax.numpy as jnp
from jax import lax
from jax.experimental import pallas as pl
from jax.experimental.pallas import tpu as pltpu
from jax.experimental.pallas import tpu_sc as plsc

N = 10000
E = 320000
NP = 10240
NC, NS = 2, 16
NW = NC * NS
CH = 128
NBUF = 4
NCH = 80
EPAD = NW * NCH * CH
EPW = 2 * E // NW
DROWS = 2 * NP // 16
RB = 2048
_MESH = plsc.VectorSubcoreMesh(core_axis_name="c", subcore_axis_name="s")


@functools.partial(
    pl.kernel,
    out_type=jax.ShapeDtypeStruct((NW * 2 * NP,), jnp.float32),
    mesh=_MESH,
    scratch_types=[
        pltpu.VMEM((EPW,), jnp.int32),
        pltpu.VMEM((2 * NP,), jnp.float32),
    ],
    compiler_params=pltpu.CompilerParams(needs_layout_passes=False),
)
def _deg_kernel(e_hbm, z_hbm, out_hbm, eidx, acc):
    cid = lax.axis_index("c")
    sid = lax.axis_index("s")
    wid = sid * NC + cid
    pltpu.sync_copy(e_hbm.at[pl.ds(wid * EPW, EPW)], eidx)
    pltpu.sync_copy(z_hbm, acc)

    off = jnp.where(wid >= NS, NP, 0).astype(jnp.int32)
    ones = jnp.ones((16,), jnp.float32)

    def body(i, c):
        for u in range(4):
            ev = eidx[pl.ds(i * 64 + u * 16, 16)] + off
            plsc.addupdate_scatter(acc, [ev], ones)
        return c

    lax.fori_loop(0, EPW // 64, body, 0)
    pltpu.sync_copy(acc, out_hbm.at[pl.ds(wid * 2 * NP, 2 * NP)])


def _make_spmm(D=64):
    rpt = NP // NS
    npairs = NCH // (2 * NBUF)

    @functools.partial(
        pl.kernel,
        out_type=jax.ShapeDtypeStruct((NC * NP, D), jnp.float32),
        mesh=_MESH,
        scratch_types=[
            pltpu.VMEM((NCH, CH), jnp.int32),
            pltpu.VMEM((NCH, CH), jnp.int32),
            pltpu.VMEM((2 * NBUF, CH, D), jnp.float32),
            pltpu.VMEM_SHARED((NP, D), jnp.float32),
            pltpu.SemaphoreType.DMA,
            pltpu.SemaphoreType.DMA,
            pltpu.SemaphoreType.DMA,
            pltpu.SemaphoreType.DMA,
        ],
        compiler_params=pltpu.CompilerParams(use_tc_tiling_on_sc=False),
    )
    def spmm(y_hbm, src_hbm, dst_hbm, out_hbm, src_v, dst_v, bufs, acc,
             gsA, gsB, ssA, ssB):
        cid = lax.axis_index("c")
        sid = lax.axis_index("s")
        wid = sid * NC + cid
        pltpu.sync_copy(src_hbm.at[pl.ds(wid * NCH, NCH)], src_v)
        pltpu.sync_copy(dst_hbm.at[pl.ds(wid * NCH, NCH)], dst_v)

        def gather(j, b, sem):
            pltpu.async_copy(y_hbm.at[src_v.at[j]], bufs.at[b], sem)

        def gwait(b, sem):
            pltpu.make_async_copy(y_hbm.at[src_v.at[0]], bufs.at[b], sem).wait()

        def scat(j, b, sem):
            pltpu.async_copy(bufs.at[b], acc.at[dst_v.at[j]], sem, add=True)

        def swait(b, sem):
            pltpu.make_async_copy(bufs.at[b], acc.at[dst_v.at[0]], sem).wait()

        for b in range(NBUF):
            gather(b, b, gsA)

        def zrow(r, c):
            for j in range(D // 16):
                bufs[NBUF, r, pl.ds(j * 16, 16)] = jnp.zeros((16,), jnp.float32)
            return c

        lax.fori_loop(0, CH, zrow, 0)
        for q in range(rpt // CH):
            pltpu.async_copy(bufs.at[NBUF], acc.at[pl.ds(sid * rpt + q * CH, CH)], ssA)
        for q in range(rpt // CH):
            pltpu.make_async_copy(bufs.at[NBUF], acc.at[pl.ds(sid * rpt, CH)], ssA).wait()
        plsc.subcore_barrier()

        def pair(gg, c):
            jA = gg * 2 * NBUF
            jB = jA + NBUF
            for b in range(NBUF):
                gather(jB + b, NBUF + b, gsB)
            for b in range(NBUF):
                gwait(b, gsA)
                scat(jA + b, b, ssA)
            for b in range(NBUF):
                swait(b, ssA)

            @pl.when(gg < npairs - 1)
            def _():
                for b in range(NBUF):
                    gather(jA + 2 * NBUF + b, b, gsA)

            for b in range(NBUF):
                gwait(NBUF + b, gsB)
                scat(jB + b, NBUF + b, ssB)
            for b in range(NBUF):
                swait(NBUF + b, ssB)
            return c

        lax.fori_loop(0, npairs, pair, 0)
        plsc.subcore_barrier()
        for q in range(rpt // CH):
            r0 = sid * rpt + q * CH
            pltpu.sync_copy(acc.at[pl.ds(r0, CH)], out_hbm.at[pl.ds(cid * NP + r0, CH)])

    return spmm


_spmm64 = _make_spmm(64)


def _norm(deg):
    return jnp.where(deg > 0, lax.rsqrt(jnp.maximum(deg, 1.0)), 0.0)


def _b_body(x_ref, w_ref, po_ref, o_ref):
    ns = _norm(jnp.sum(po_ref[...], axis=0))
    o_ref[...] = jnp.dot(
        x_ref[...], w_ref[...], preferred_element_type=jnp.float32
    ) * ns[:, None]


def _tc_proj1(xp, W1, p_out):
    return pl.pallas_call(
        _b_body,
        grid=(NP // RB,),
        in_specs=[
            pl.BlockSpec((RB, 128), lambda i: (i, 0)),
            pl.BlockSpec((128, 128), lambda i: (0, 0)),
            pl.BlockSpec((NW, RB), lambda i: (0, i)),
        ],
        out_specs=pl.BlockSpec((RB, 128), lambda i: (i, 0)),
        out_shape=jax.ShapeDtypeStruct((NP, 128), jnp.float32),
    )(xp, W1, p_out)


def _c_body(a0a_ref, a1a_ref, a0b_ref, a1b_ref, pi_ref, po_ref, b1_ref, w2_ref,
            o_ref, nin_ref):
    nin = _norm(jnp.sum(pi_ref[...], axis=0))
    nout = _norm(jnp.sum(po_ref[...], axis=0))
    agg = jnp.concatenate(
        [a0a_ref[...] + a1a_ref[...], a0b_ref[...] + a1b_ref[...]], axis=1
    )
    h = jnp.maximum(agg * nin[:, None] + b1_ref[...], 0.0)
    o_ref[...] = jnp.dot(
        h * nout[:, None], w2_ref[...], preferred_element_type=jnp.float32
    )
    nin_ref[...] = nin[:, None]


def _tc_mid(agg1a, agg1b, p_in, p_out, b1, W2):
    return pl.pallas_call(
        _c_body,
        grid=(NP // RB,),
        in_specs=[
            pl.BlockSpec((RB, 64), lambda i: (i, 0)),
            pl.BlockSpec((RB, 64), lambda i: (i + NP // RB, 0)),
            pl.BlockSpec((RB, 64), lambda i: (i, 0)),
            pl.BlockSpec((RB, 64), lambda i: (i + NP // RB, 0)),
            pl.BlockSpec((NW, RB), lambda i: (0, i)),
            pl.BlockSpec((NW, RB), lambda i: (0, i)),
            pl.BlockSpec((1, 128), lambda i: (0, 0)),
            pl.BlockSpec((128, 64), lambda i: (0, 0)),
        ],
        out_specs=[
            pl.BlockSpec((RB, 64), lambda i: (i, 0)),
            pl.BlockSpec((RB, 1), lambda i: (i, 0)),
        ],
        out_shape=[
            jax.ShapeDtypeStruct((NP, 64), jnp.float32),
            jax.ShapeDtypeStruct((NP, 1), jnp.float32),
        ],
    )(agg1a, agg1a, agg1b, agg1b, p_in, p_out, b1, W2)


def _d_body(a0_ref, a1_ref, nin_ref, b2_ref, o_ref):
    z = (a0_ref[...] + a1_ref[...]) * nin_ref[...] + b2_ref[...]
    z = z - jnp.max(z, axis=1, keepdims=True)
    ez = jnp.exp(z)
    o_ref[...] = ez / jnp.sum(ez, axis=1, keepdims=True)


def _tc_out(a0, a1, nin2, b2):
    rb = 1000
    return pl.pallas_call(
        _d_body,
        grid=(N // rb,),
        in_specs=[
            pl.BlockSpec((rb, 64), lambda i: (i, 0)),
            pl.BlockSpec((rb, 64), lambda i: (i, 0)),
            pl.BlockSpec((rb, 1), lambda i: (i, 0)),
            pl.BlockSpec((1, 64), lambda i: (0, 0)),
        ],
        out_specs=pl.BlockSpec((rb, 64), lambda i: (i, 0)),
        out_shape=jax.ShapeDtypeStruct((N, 64), jnp.float32),
    )(a0, a1, nin2, b2)


def kernel(inputs, edge_index, W1, b1, W2, b2):
    x = inputs
    eflat = edge_index.reshape(-1)
    pad = EPAD - E
    pad_src = (jnp.arange(pad, dtype=jnp.int32) % N).astype(jnp.int32)
    pad_dst = (N + jnp.arange(pad, dtype=jnp.int32) % (NP - N)).astype(jnp.int32)
    srcp = jnp.concatenate([edge_index[0], pad_src]).reshape(NW * NCH, CH)
    dstp = jnp.concatenate([edge_index[1], pad_dst]).reshape(NW * NCH, CH)
    xp = jnp.pad(x, ((0, NP - N), (0, 0)))

    parts = _deg_kernel(eflat, jnp.zeros((2 * NP,), jnp.float32))
    pflat = parts.reshape(NW, 2 * NP)
    p_out = pflat[:, :NP]
    p_in = pflat[:, NP:]

    y1 = _tc_proj1(xp, W1, p_out)
    agg1a = _spmm64(y1[:, :64], srcp, dstp)
    agg1b = _spmm64(y1[:, 64:], srcp, dstp)
    y2, nin2 = _tc_mid(agg1a, agg1b, p_in, p_out, b1.reshape(1, 128), W2)
    agg2 = _spmm64(y2, srcp, dstp)
    return _tc_out(agg2[:N], agg2[NP:NP + N], nin2[:N], b2.reshape(1, 64))


# speedup vs baseline: 1.0360x; 1.0360x over previous
import functools

import jax
import jax.numpy as jnp
from jax import lax
from jax.experimental import pallas as pl
from jax.experimental.pallas import tpu as pltpu
from jax.experimental.pallas import tpu_sc as plsc

N = 10000
E = 320000
NP = 10240
NC, NS = 2, 16
NW = NC * NS
CH = 128
NBUF = 4
NCH = 80
EPAD = NW * NCH * CH
EPW = 2 * E // NW
DROWS = 2 * NP // 16
RB = 2048
_MESH = plsc.VectorSubcoreMesh(core_axis_name="c", subcore_axis_name="s")


@functools.partial(
    pl.kernel,
    out_type=jax.ShapeDtypeStruct((NW * 2 * NP,), jnp.float32),
    mesh=_MESH,
    scratch_types=[
        pltpu.VMEM((EPW,), jnp.int32),
        pltpu.VMEM((2 * NP,), jnp.float32),
    ],
    compiler_params=pltpu.CompilerParams(needs_layout_passes=False),
)
def _deg_kernel(e_hbm, z_hbm, out_hbm, eidx, acc):
    cid = lax.axis_index("c")
    sid = lax.axis_index("s")
    wid = sid * NC + cid
    pltpu.sync_copy(e_hbm.at[pl.ds(wid * EPW, EPW)], eidx)
    pltpu.sync_copy(z_hbm, acc)

    off = jnp.where(wid >= NS, NP, 0).astype(jnp.int32)
    ones = jnp.ones((16,), jnp.float32)

    def body(i, c):
        for u in range(4):
            ev = eidx[pl.ds(i * 64 + u * 16, 16)] + off
            plsc.addupdate_scatter(acc, [ev], ones)
        return c

    lax.fori_loop(0, EPW // 64, body, 0)
    pltpu.sync_copy(acc, out_hbm.at[pl.ds(wid * 2 * NP, 2 * NP)])


def _make_spmm(D=64):
    rpt = NP // NS
    npairs = NCH // (2 * NBUF)

    @functools.partial(
        pl.kernel,
        out_type=jax.ShapeDtypeStruct((NC * NP, D), jnp.float32),
        mesh=_MESH,
        scratch_types=[
            pltpu.VMEM((NCH, CH), jnp.int32),
            pltpu.VMEM((NCH, CH), jnp.int32),
            pltpu.VMEM((2 * NBUF, CH, D), jnp.float32),
            pltpu.VMEM_SHARED((NP, D), jnp.float32),
            pltpu.SemaphoreType.DMA,
            pltpu.SemaphoreType.DMA,
            pltpu.SemaphoreType.DMA,
            pltpu.SemaphoreType.DMA,
        ],
        compiler_params=pltpu.CompilerParams(use_tc_tiling_on_sc=False),
    )
    def spmm(y_hbm, src_hbm, dst_hbm, out_hbm, src_v, dst_v, bufs, acc,
             gsA, gsB, ssA, ssB):
        cid = lax.axis_index("c")
        sid = lax.axis_index("s")
        wid = sid * NC + cid
        pltpu.sync_copy(src_hbm.at[pl.ds(wid * NCH, NCH)], src_v)
        pltpu.sync_copy(dst_hbm.at[pl.ds(wid * NCH, NCH)], dst_v)

        def gather(j, b, sem):
            pltpu.async_copy(y_hbm.at[src_v.at[j]], bufs.at[b], sem)

        def gwait(b, sem):
            pltpu.make_async_copy(y_hbm.at[src_v.at[0]], bufs.at[b], sem).wait()

        def scat(j, b, sem):
            pltpu.async_copy(bufs.at[b], acc.at[dst_v.at[j]], sem, add=True)

        def swait(b, sem):
            pltpu.make_async_copy(bufs.at[b], acc.at[dst_v.at[0]], sem).wait()

        for b in range(NBUF):
            gather(b, b, gsA)

        def zrow(r, c):
            for j in range(D // 16):
                bufs[NBUF, r, pl.ds(j * 16, 16)] = jnp.zeros((16,), jnp.float32)
            return c

        lax.fori_loop(0, CH, zrow, 0)
        for q in range(rpt // CH):
            pltpu.async_copy(bufs.at[NBUF], acc.at[pl.ds(sid * rpt + q * CH, CH)], ssA)
        for q in range(rpt // CH):
            pltpu.make_async_copy(bufs.at[NBUF], acc.at[pl.ds(sid * rpt, CH)], ssA).wait()
        plsc.subcore_barrier()

        def pair(gg, c):
            jA = gg * 2 * NBUF
            jB = jA + NBUF
            for b in range(NBUF):
                gather(jB + b, NBUF + b, gsB)
            for b in range(NBUF):
                gwait(b, gsA)
                scat(jA + b, b, ssA)
            for b in range(NBUF):
                swait(b, ssA)

            @pl.when(gg < npairs - 1)
            def _():
                for b in range(NBUF):
                    gather(jA + 2 * NBUF + b, b, gsA)

            for b in range(NBUF):
                gwait(NBUF + b, gsB)
                scat(jB + b, NBUF + b, ssB)
            for b in range(NBUF):
                swait(NBUF + b, ssB)
            return c

        lax.fori_loop(0, npairs, pair, 0)
        plsc.subcore_barrier()
        for q in range(rpt // CH):
            r0 = sid * rpt + q * CH
            pltpu.sync_copy(acc.at[pl.ds(r0, CH)], out_hbm.at[pl.ds(cid * NP + r0, CH)])

    return spmm


_spmm64 = _make_spmm(64)


def _norm(deg):
    return jnp.where(deg > 0, lax.rsqrt(jnp.maximum(deg, 1.0)), 0.0)


def _b_body(x_ref, w_ref, po_ref, o_ref):
    ns = _norm(jnp.sum(po_ref[...], axis=0))
    o_ref[...] = jnp.dot(
        x_ref[...], w_ref[...], preferred_element_type=jnp.float32
    ) * ns[:, None]


def _tc_proj1(xp, W1, p_out):
    return pl.pallas_call(
        _b_body,
        grid=(NP // RB,),
        in_specs=[
            pl.BlockSpec((RB, 128), lambda i: (i, 0)),
            pl.BlockSpec((128, 128), lambda i: (0, 0)),
            pl.BlockSpec((NW, RB), lambda i: (0, i)),
        ],
        out_specs=pl.BlockSpec((RB, 128), lambda i: (i, 0)),
        out_shape=jax.ShapeDtypeStruct((NP, 128), jnp.float32),
    )(xp, W1, p_out)


def _c_body(a0a_ref, a1a_ref, a0b_ref, a1b_ref, pi_ref, po_ref, b1_ref, w2_ref,
            o_ref, nin_ref):
    nin = _norm(jnp.sum(pi_ref[...], axis=0))
    nout = _norm(jnp.sum(po_ref[...], axis=0))
    agg = jnp.concatenate(
        [a0a_ref[...] + a1a_ref[...], a0b_ref[...] + a1b_ref[...]], axis=1
    )
    h = jnp.maximum(agg * nin[:, None] + b1_ref[...], 0.0)
    o_ref[...] = jnp.dot(
        h * nout[:, None], w2_ref[...], preferred_element_type=jnp.float32
    )
    nin_ref[...] = nin[:, None]


def _tc_mid(agg1a, agg1b, p_in, p_out, b1, W2):
    return pl.pallas_call(
        _c_body,
        grid=(NP // RB,),
        in_specs=[
            pl.BlockSpec((RB, 64), lambda i: (i, 0)),
            pl.BlockSpec((RB, 64), lambda i: (i + NP // RB, 0)),
            pl.BlockSpec((RB, 64), lambda i: (i, 0)),
            pl.BlockSpec((RB, 64), lambda i: (i + NP // RB, 0)),
            pl.BlockSpec((NW, RB), lambda i: (0, i)),
            pl.BlockSpec((NW, RB), lambda i: (0, i)),
            pl.BlockSpec((1, 128), lambda i: (0, 0)),
            pl.BlockSpec((128, 64), lambda i: (0, 0)),
        ],
        out_specs=[
            pl.BlockSpec((RB, 64), lambda i: (i, 0)),
            pl.BlockSpec((RB, 1), lambda i: (i, 0)),
        ],
        out_shape=[
            jax.ShapeDtypeStruct((NP, 64), jnp.float32),
            jax.ShapeDtypeStruct((NP, 1), jnp.float32),
        ],
    )(agg1a, agg1a, agg1b, agg1b, p_in, p_out, b1, W2)


def _d_body(a0_ref, a1_ref, nin_ref, b2_ref, o_ref):
    z = (a0_ref[...] + a1_ref[...]) * nin_ref[...] + b2_ref[...]
    z = z - jnp.max(z, axis=1, keepdims=True)
    ez = jnp.exp(z)
    o_ref[...] = ez / jnp.sum(ez, axis=1, keepdims=True)


def _tc_out(agg2, nin2, b2):
    return pl.pallas_call(
        _d_body,
        grid=(NP // RB,),
        in_specs=[
            pl.BlockSpec((RB, 64), lambda i: (i, 0)),
            pl.BlockSpec((RB, 64), lambda i: (i + NP // RB, 0)),
            pl.BlockSpec((RB, 1), lambda i: (i, 0)),
            pl.BlockSpec((1, 64), lambda i: (0, 0)),
        ],
        out_specs=pl.BlockSpec((RB, 64), lambda i: (i, 0)),
        out_shape=jax.ShapeDtypeStruct((NP, 64), jnp.float32),
    )(agg2, agg2, nin2, b2)


def kernel(inputs, edge_index, W1, b1, W2, b2):
    x = inputs
    eflat = edge_index.reshape(-1)
    pad = EPAD - E
    pad_src = (jnp.arange(pad, dtype=jnp.int32) % N).astype(jnp.int32)
    pad_dst = (N + jnp.arange(pad, dtype=jnp.int32) % (NP - N)).astype(jnp.int32)
    srcp = jnp.concatenate([edge_index[0], pad_src]).reshape(NW * NCH, CH)
    dstp = jnp.concatenate([edge_index[1], pad_dst]).reshape(NW * NCH, CH)
    xp = jnp.pad(x, ((0, NP - N), (0, 0)))

    parts = _deg_kernel(eflat, jnp.zeros((2 * NP,), jnp.float32))
    pflat = parts.reshape(NW, 2 * NP)
    p_out = pflat[:, :NP]
    p_in = pflat[:, NP:]

    y1 = _tc_proj1(xp, W1, p_out)
    agg1a = _spmm64(y1[:, :64], srcp, dstp)
    agg1b = _spmm64(y1[:, 64:], srcp, dstp)
    y2, nin2 = _tc_mid(agg1a, agg1b, p_in, p_out, b1.reshape(1, 128), W2)
    agg2 = _spmm64(y2, srcp, dstp)
    out = _tc_out(agg2, nin2, b2.reshape(1, 64))
    return out[:N]


# speedup vs baseline: 1.0440x; 1.0078x over previous
import functools

import jax
import jax.numpy as jnp
from jax import lax
from jax.experimental import pallas as pl
from jax.experimental.pallas import tpu as pltpu
from jax.experimental.pallas import tpu_sc as plsc

N = 10000
E = 320000
NP = 10240
NC, NS = 2, 16
NW = NC * NS
CH = 128
NBUF = 4
NCH = 80
EPAD = NW * NCH * CH
EPW = 2 * E // NW
DROWS = 2 * NP // 16
RB = 2048
_MESH = plsc.VectorSubcoreMesh(core_axis_name="c", subcore_axis_name="s")


@functools.partial(
    pl.kernel,
    out_type=jax.ShapeDtypeStruct((NW * 2 * NP,), jnp.float32),
    mesh=_MESH,
    scratch_types=[
        pltpu.VMEM((EPW,), jnp.int32),
        pltpu.VMEM((2 * NP,), jnp.float32),
    ],
    compiler_params=pltpu.CompilerParams(needs_layout_passes=False),
)
def _deg_kernel(e_hbm, z_hbm, out_hbm, eidx, acc):
    cid = lax.axis_index("c")
    sid = lax.axis_index("s")
    wid = sid * NC + cid
    pltpu.sync_copy(e_hbm.at[pl.ds(wid * EPW, EPW)], eidx)
    pltpu.sync_copy(z_hbm, acc)

    off = jnp.where(wid >= NS, NP, 0).astype(jnp.int32)
    ones = jnp.ones((16,), jnp.float32)

    def body(i, c):
        for u in range(4):
            ev = eidx[pl.ds(i * 64 + u * 16, 16)] + off
            plsc.addupdate_scatter(acc, [ev], ones)
        return c

    lax.fori_loop(0, EPW // 64, body, 0)
    pltpu.sync_copy(acc, out_hbm.at[pl.ds(wid * 2 * NP, 2 * NP)])


def _make_spmm(D=64):
    rpt = NP // NS
    npairs = NCH // (2 * NBUF)

    @functools.partial(
        pl.kernel,
        out_type=jax.ShapeDtypeStruct((NC * NP, D), jnp.float32),
        mesh=_MESH,
        scratch_types=[
            pltpu.VMEM((NCH, CH), jnp.int32),
            pltpu.VMEM((NCH, CH), jnp.int32),
            pltpu.VMEM((2 * NBUF, CH, D), jnp.float32),
            pltpu.VMEM_SHARED((NP, D), jnp.float32),
            pltpu.SemaphoreType.DMA,
            pltpu.SemaphoreType.DMA,
            pltpu.SemaphoreType.DMA,
            pltpu.SemaphoreType.DMA,
        ],
        compiler_params=pltpu.CompilerParams(use_tc_tiling_on_sc=False),
    )
    def spmm(y_hbm, src_hbm, dst_hbm, out_hbm, src_v, dst_v, bufs, acc,
             gsA, gsB, ssA, ssB):
        cid = lax.axis_index("c")
        sid = lax.axis_index("s")
        wid = sid * NC + cid
        pltpu.sync_copy(src_hbm.at[pl.ds(wid * NCH, NCH)], src_v)
        pltpu.sync_copy(dst_hbm.at[pl.ds(wid * NCH, NCH)], dst_v)

        def gather(j, b, sem):
            pltpu.async_copy(y_hbm.at[src_v.at[j]], bufs.at[b], sem)

        def gwait(b, sem):
            pltpu.make_async_copy(y_hbm.at[src_v.at[0]], bufs.at[b], sem).wait()

        def scat(j, b, sem):
            pltpu.async_copy(bufs.at[b], acc.at[dst_v.at[j]], sem, add=True)

        def swait(b, sem):
            pltpu.make_async_copy(bufs.at[b], acc.at[dst_v.at[0]], sem).wait()

        for b in range(NBUF):
            gather(b, b, gsA)

        def zrow(r, c):
            for j in range(D // 16):
                bufs[NBUF, r, pl.ds(j * 16, 16)] = jnp.zeros((16,), jnp.float32)
            return c

        lax.fori_loop(0, CH, zrow, 0)
        for q in range(rpt // CH):
            pltpu.async_copy(bufs.at[NBUF], acc.at[pl.ds(sid * rpt + q * CH, CH)], ssA)
        for q in range(rpt // CH):
            pltpu.make_async_copy(bufs.at[NBUF], acc.at[pl.ds(sid * rpt, CH)], ssA).wait()
        plsc.subcore_barrier()

        def pair(gg, c):
            jA = gg * 2 * NBUF
            jB = jA + NBUF
            for b in range(NBUF):
                gather(jB + b, NBUF + b, gsB)
            for b in range(NBUF):
                gwait(b, gsA)
                scat(jA + b, b, ssA)
            for b in range(NBUF):
                swait(b, ssA)

            @pl.when(gg < npairs - 1)
            def _():
                for b in range(NBUF):
                    gather(jA + 2 * NBUF + b, b, gsA)

            for b in range(NBUF):
                gwait(NBUF + b, gsB)
                scat(jB + b, NBUF + b, ssB)
            for b in range(NBUF):
                swait(NBUF + b, ssB)
            return c

        lax.fori_loop(0, npairs, pair, 0)
        plsc.subcore_barrier()
        for q in range(rpt // CH):
            r0 = sid * rpt + q * CH
            pltpu.sync_copy(acc.at[pl.ds(r0, CH)], out_hbm.at[pl.ds(cid * NP + r0, CH)])

    return spmm


_spmm64 = _make_spmm(64)


def _norm(deg):
    return jnp.where(deg > 0, lax.rsqrt(jnp.maximum(deg, 1.0)), 0.0)


def _b_body(x_ref, w_ref, po_ref, o_ref):
    ns = _norm(jnp.sum(po_ref[...], axis=0))
    o_ref[...] = jnp.dot(
        x_ref[...], w_ref[...], preferred_element_type=jnp.float32
    ) * ns[:, None]


def _tc_proj1(xp, W1, p_out):
    return pl.pallas_call(
        _b_body,
        grid=(NP // RB,),
        in_specs=[
            pl.BlockSpec((RB, 128), lambda i: (i, 0)),
            pl.BlockSpec((128, 128), lambda i: (0, 0)),
            pl.BlockSpec((NW, RB), lambda i: (0, i)),
        ],
        out_specs=pl.BlockSpec((RB, 128), lambda i: (i, 0)),
        out_shape=jax.ShapeDtypeStruct((NP, 128), jnp.float32),
    )(xp, W1, p_out)


def _c_body(a0a_ref, a1a_ref, a0b_ref, a1b_ref, pi_ref, po_ref, b1_ref, w2_ref,
            o_ref):
    nin = _norm(jnp.sum(pi_ref[...], axis=0))
    nout = _norm(jnp.sum(po_ref[...], axis=0))
    agg = jnp.concatenate(
        [a0a_ref[...] + a1a_ref[...], a0b_ref[...] + a1b_ref[...]], axis=1
    )
    h = jnp.maximum(agg * nin[:, None] + b1_ref[...], 0.0)
    o_ref[...] = jnp.dot(
        h * nout[:, None], w2_ref[...], preferred_element_type=jnp.float32
    )


def _tc_mid(agg1a, agg1b, p_in, p_out, b1, W2):
    return pl.pallas_call(
        _c_body,
        grid=(NP // RB,),
        in_specs=[
            pl.BlockSpec((RB, 64), lambda i: (i, 0)),
            pl.BlockSpec((RB, 64), lambda i: (i + NP // RB, 0)),
            pl.BlockSpec((RB, 64), lambda i: (i, 0)),
            pl.BlockSpec((RB, 64), lambda i: (i + NP // RB, 0)),
            pl.BlockSpec((NW, RB), lambda i: (0, i)),
            pl.BlockSpec((NW, RB), lambda i: (0, i)),
            pl.BlockSpec((1, 128), lambda i: (0, 0)),
            pl.BlockSpec((128, 64), lambda i: (0, 0)),
        ],
        out_specs=pl.BlockSpec((RB, 64), lambda i: (i, 0)),
        out_shape=jax.ShapeDtypeStruct((NP, 64), jnp.float32),
    )(agg1a, agg1a, agg1b, agg1b, p_in, p_out, b1, W2)


def _d_body(a0_ref, a1_ref, pi_ref, b2_ref, o_ref):
    nin = _norm(jnp.sum(pi_ref[...], axis=0))
    z = (a0_ref[...] + a1_ref[...]) * nin[:, None] + b2_ref[...]
    z = z - jnp.max(z, axis=1, keepdims=True)
    ez = jnp.exp(z)
    o_ref[...] = ez / jnp.sum(ez, axis=1, keepdims=True)


def _tc_out(agg2, p_in, b2):
    return pl.pallas_call(
        _d_body,
        grid=(NP // RB,),
        in_specs=[
            pl.BlockSpec((RB, 64), lambda i: (i, 0)),
            pl.BlockSpec((RB, 64), lambda i: (i + NP // RB, 0)),
            pl.BlockSpec((NW, RB), lambda i: (0, i)),
            pl.BlockSpec((1, 64), lambda i: (0, 0)),
        ],
        out_specs=pl.BlockSpec((RB, 64), lambda i: (i, 0)),
        out_shape=jax.ShapeDtypeStruct((NP, 64), jnp.float32),
    )(agg2, agg2, p_in, b2)


def kernel(inputs, edge_index, W1, b1, W2, b2):
    x = inputs
    eflat = edge_index.reshape(-1)
    pad = EPAD - E
    pad_src = (jnp.arange(pad, dtype=jnp.int32) % N).astype(jnp.int32)
    pad_dst = (N + jnp.arange(pad, dtype=jnp.int32) % (NP - N)).astype(jnp.int32)
    srcp = jnp.concatenate([edge_index[0], pad_src]).reshape(NW * NCH, CH)
    dstp = jnp.concatenate([edge_index[1], pad_dst]).reshape(NW * NCH, CH)
    xp = jnp.pad(x, ((0, NP - N), (0, 0)))

    parts = _deg_kernel(eflat, jnp.zeros((2 * NP,), jnp.float32))
    pflat = parts.reshape(NW, 2 * NP)
    p_out = pflat[:, :NP]
    p_in = pflat[:, NP:]

    y1 = _tc_proj1(xp, W1, p_out)
    agg1a = _spmm64(y1[:, :64], srcp, dstp)
    agg1b = _spmm64(y1[:, 64:], srcp, dstp)
    y2 = _tc_mid(agg1a, agg1b, p_in, p_out, b1.reshape(1, 128), W2)
    agg2 = _spmm64(y2, srcp, dstp)
    out = _tc_out(agg2, p_in, b2.reshape(1, 64))
    return out[:N]
